# baseline (speedup 1.0000x reference)
import dataclasses

import jax
import jax.numpy as jnp
import numpy as np
from jax import lax
from jax.experimental import pallas as pl
from jax.experimental.pallas import tpu as pltpu
from jax.experimental.pallas import tpu_sc as plsc

_B, _P, _DIN, _H, _K = 256, 256, 4, 64, 16
_P1 = _P // 2
_G = 4
_BIG = np.float32(3e38)
_HIGH = jax.lax.Precision.HIGHEST
_NT = (((1,), (1,)), ((), ()))
_TN = (((0,), (0,)), ((), ()))
_NSC, _NTILE = 2, 16
_GPT = _B // (_NSC * _NTILE)


def _split3(h):
    bf = jnp.bfloat16
    hi = h.astype(bf)
    r1 = h - hi.astype(jnp.float32)
    mid = r1.astype(bf)
    lo = (r1 - mid.astype(jnp.float32)).astype(bf)
    return hi, mid, lo


def _dmat(h, P):
    f32 = np.float32
    hb = h.astype(jnp.bfloat16)
    hh = jax.lax.dot_general(hb, hb, _NT, preferred_element_type=f32)
    sq = jnp.sum(h * h, axis=1, keepdims=True)
    eye = (jax.lax.broadcasted_iota(jnp.int32, (P, P), 0)
           == jax.lax.broadcasted_iota(jnp.int32, (P, P), 1)).astype(f32)
    sq_row = jax.lax.dot_general(sq, eye, _TN, preferred_element_type=f32,
                                 precision=_HIGH)
    return (sq + sq_row) - 2.0 * hh, hb


def _edge_body(h, hb, P, W1a, W1b, b1, W2, b2, sels):
    f32, bf = np.float32, jnp.bfloat16
    t1 = jax.lax.dot_general(hb, W1a, (((1,), (0,)), ((), ())),
                             preferred_element_type=f32)
    g_hi, g_mid, g_lo = _split3(h)
    S = jnp.zeros((P, _H), f32)
    for oh in sels:
        G = (jax.lax.dot_general(oh, g_hi, _TN, preferred_element_type=f32)
             + jax.lax.dot_general(oh, g_mid, _TN, preferred_element_type=f32)
             + jax.lax.dot_general(oh, g_lo, _TN, preferred_element_type=f32))
        dk = (G - h).astype(bf)
        m = jnp.maximum(
            (t1 + jnp.dot(dk, W1b, preferred_element_type=f32)) + b1, 0.0)
        S = S + jnp.dot(m.astype(bf), W2, preferred_element_type=f32)
    agg = S + f32(_K) * b2
    return jnp.max(agg.reshape(P // 2, 2, _H), axis=1)


def _edge_multi(hs, P, W1a, W1b, b1, W2, b2):
    f32, bf = np.float32, jnp.bfloat16
    fiota = jax.lax.broadcasted_iota(jnp.int32, (P, P), 0).astype(f32)
    st = []
    for h in hs:
        D, hb = _dmat(h, P)
        st.append({"h": h, "hb": hb, "D": D, "sels": []})
    for _ in range(_K):
        for s in st:
            colmin = jnp.min(s["D"], axis=0, keepdims=True)
            cand = jnp.where(s["D"] == colmin, fiota, f32(P))
            minidx = jnp.min(cand, axis=0, keepdims=True)
            sel = fiota == minidx
            s["sels"].append(sel.astype(bf))
            s["D"] = jnp.where(sel, _BIG, s["D"])
    return [_edge_body(s["h"], s["hb"], P, W1a, W1b, b1, W2, b2, s["sels"])
            for s in st]


def _kernel_a(x_ref, W_in_ref, b_in_ref, W1a0_ref, W1b0_ref, b10_ref,
              W20_ref, b20_ref, h1_ref, d1_ref):
    f32 = np.float32
    hs = []
    for i in range(_G):
        x = x_ref[i].astype(jnp.bfloat16)
        hs.append(jnp.maximum(
            jnp.dot(x, W_in_ref[...], preferred_element_type=f32)
            + b_in_ref[...], 0.0))
    hs = _edge_multi(hs, _P, W1a0_ref[...], W1b0_ref[...], b10_ref[...],
                     W20_ref[...], b20_ref[...])
    for i in range(_G):
        h1_ref[i] = hs[i]
        d1_ref[i], _ = _dmat(hs[i], _P1)


def _sc_topk(d_hbm, o_hbm, kbuf, obuf, sem):
    i32 = jnp.int32
    wid = lax.axis_index("c") * _NTILE + lax.axis_index("s")
    lane = lax.iota(i32, 16)

    def merge(a, av, b, bv):
        br = jnp.flip(b)
        brv = jnp.flip(bv)
        mk = jnp.minimum(a, br)
        mv = jnp.where(a <= br, av, brv)
        return plsc.sort_key_val(mk, mv)

    for gi in range(_GPT):
        g = wid * _GPT + gi
        pltpu.async_copy(d_hbm.at[g], kbuf, sem).wait()

        @pl.loop(0, _P1)
        def _(r):
            pairs = []
            for j in range(_P1 // 16):
                kj = kbuf[r, pl.ds(j * 16, 16)]
                pairs.append(plsc.sort_key_val(kj, lane + j * 16))
            while len(pairs) > 1:
                pairs = [merge(*pairs[i], *pairs[i + 1])
                         for i in range(0, len(pairs), 2)]
            _, vals = pairs[0]
            plsc.store_scatter(obuf, [lane, jnp.full((16,), r, i32)], vals)

        pltpu.async_copy(obuf, o_hbm.at[g], sem).wait()


def _kernel_b(h1_ref, idx_ref, W1a1_ref, W1b1_ref, b11_ref, W21_ref,
              b21_ref, g_ref):
    f32, bf = np.float32, jnp.bfloat16
    fiota = jax.lax.broadcasted_iota(jnp.int32, (_P1, _P1), 0)
    for i in range(_G):
        h = h1_ref[i]
        idx = idx_ref[i]
        sels = [(fiota == idx[k:k + 1, :]).astype(bf) for k in range(_K)]
        hp = _edge_body(h, h.astype(bf), _P1, W1a1_ref[...], W1b1_ref[...],
                        b11_ref[...], W21_ref[...], b21_ref[...], sels)
        g_ref[i] = jnp.max(hp, axis=0, keepdims=True)


def _out_kernel(g_ref, gx_ref, Wg_ref, Wx_ref, bo1_ref, Wo2_ref, bo2_ref,
                Wo3_ref, bo3_ref, o_ref):
    f32, bf = np.float32, jnp.bfloat16
    t = (jnp.dot(g_ref[...].astype(bf), Wg_ref[...], preferred_element_type=f32)
         + jnp.dot(gx_ref[...].astype(bf), Wx_ref[...], preferred_element_type=f32)
         + bo1_ref[...])
    t = jnp.maximum(t, 0.0)
    t = jnp.maximum(
        jnp.dot(t.astype(bf), Wo2_ref[...], preferred_element_type=f32)
        + bo2_ref[...], 0.0)
    o_ref[...] = (jnp.dot(t.astype(bf), Wo3_ref[...], preferred_element_type=f32)
                  + bo3_ref[...])


def _sc_compiler_params():
    cp = pltpu.CompilerParams()
    if "needs_layout_passes" in pltpu.CompilerParams.__dataclass_fields__:
        cp = dataclasses.replace(cp, needs_layout_passes=False)
    return cp


def kernel(x, gx, W_in, b_in, W1_0, b1_0, W2_0, b2_0, W1_1, b1_1, W2_1,
           b2_1, Wo1, bo1, Wo2, bo2, Wo3, bo3):
    f32, bf = np.float32, jnp.bfloat16
    row = lambda v: v.reshape(1, -1).astype(f32)
    full = lambda a: pl.BlockSpec(a.shape, (lambda nd: lambda b: (0,) * nd)(a.ndim))

    aconsts = (W_in.astype(bf), row(b_in),
               W1_0[:_H].astype(bf), W1_0[_H:].astype(bf), row(b1_0),
               W2_0.astype(bf), row(b2_0))
    h1, d1 = pl.pallas_call(
        _kernel_a,
        grid=(_B // _G,),
        in_specs=[pl.BlockSpec((_G, _P, _DIN), lambda b: (b, 0, 0))]
        + [full(a) for a in aconsts],
        out_specs=[pl.BlockSpec((_G, _P1, _H), lambda b: (b, 0, 0)),
                   pl.BlockSpec((_G, _P1, _P1), lambda b: (b, 0, 0))],
        out_shape=[jax.ShapeDtypeStruct((_B, _P1, _H), f32),
                   jax.ShapeDtypeStruct((_B, _P1, _P1), f32)],
    )(x, *aconsts)

    mesh = plsc.VectorSubcoreMesh(core_axis_name="c", subcore_axis_name="s")
    idxT = pl.kernel(
        _sc_topk,
        out_type=jax.ShapeDtypeStruct((_B, _K, _P1), jnp.int32),
        mesh=mesh,
        scratch_types=[pltpu.VMEM((_P1, _P1), f32),
                       pltpu.VMEM((_K, _P1), jnp.int32),
                       pltpu.SemaphoreType.DMA],
        compiler_params=_sc_compiler_params(),
    )(d1)

    bconsts = (W1_1[:_H].astype(bf), W1_1[_H:].astype(bf), row(b1_1),
               W2_1.astype(bf), row(b2_1))
    g = pl.pallas_call(
        _kernel_b,
        grid=(_B // _G,),
        in_specs=[pl.BlockSpec((_G, _P1, _H), lambda b: (b, 0, 0)),
                  pl.BlockSpec((_G, _K, _P1), lambda b: (b, 0, 0))]
        + [full(a) for a in bconsts],
        out_specs=pl.BlockSpec((_G, 1, _H), lambda b: (b, 0, 0)),
        out_shape=jax.ShapeDtypeStruct((_B, 1, _H), f32),
    )(h1, idxT, *bconsts)
    g = g.reshape(_B, _H)

    oconsts = (Wo1[:_H].astype(bf), Wo1[_H:].astype(bf), row(bo1),
               Wo2.astype(bf), row(bo2), Wo3.astype(bf), row(bo3))
    out = pl.pallas_call(
        _out_kernel,
        in_specs=[pl.BlockSpec((_B, _H), lambda: (0, 0)),
                  pl.BlockSpec((_B, _DIN), lambda: (0, 0))]
        + [pl.BlockSpec(a.shape, (lambda nd: lambda: (0,) * nd)(a.ndim))
           for a in oconsts],
        out_specs=pl.BlockSpec((_B, 1), lambda: (0, 0)),
        out_shape=jax.ShapeDtypeStruct((_B, 1), f32),
    )(g, gx, *oconsts)
    return out


# speedup vs baseline: 1.0189x; 1.0189x over previous
import dataclasses

import jax
import jax.numpy as jnp
import numpy as np
from jax import lax
from jax.experimental import pallas as pl
from jax.experimental.pallas import tpu as pltpu
from jax.experimental.pallas import tpu_sc as plsc

_B, _P, _DIN, _H, _K = 256, 256, 4, 64, 16
_P1 = _P // 2
_G = 4
_BIG = np.float32(3e38)
_HIGH = jax.lax.Precision.HIGHEST
_NT = (((1,), (1,)), ((), ()))
_TN = (((0,), (0,)), ((), ()))
_NSC, _NTILE = 2, 16
_NCHUNK = 2
_GPT = _B // _NCHUNK // (_NSC * _NTILE)


def _split3(h):
    bf = jnp.bfloat16
    hi = h.astype(bf)
    r1 = h - hi.astype(jnp.float32)
    mid = r1.astype(bf)
    lo = (r1 - mid.astype(jnp.float32)).astype(bf)
    return hi, mid, lo


def _dmat(h, P):
    f32 = np.float32
    hb = h.astype(jnp.bfloat16)
    hh = jax.lax.dot_general(hb, hb, _NT, preferred_element_type=f32)
    sq = jnp.sum(h * h, axis=1, keepdims=True)
    eye = (jax.lax.broadcasted_iota(jnp.int32, (P, P), 0)
           == jax.lax.broadcasted_iota(jnp.int32, (P, P), 1)).astype(f32)
    sq_row = jax.lax.dot_general(sq, eye, _TN, preferred_element_type=f32,
                                 precision=_HIGH)
    return (sq + sq_row) - 2.0 * hh, hb


def _edge_body(h, hb, P, W1a, W1b, b1, W2, b2, sels):
    f32, bf = np.float32, jnp.bfloat16
    t1 = jax.lax.dot_general(hb, W1a, (((1,), (0,)), ((), ())),
                             preferred_element_type=f32)
    g_hi, g_mid, g_lo = _split3(h)
    S = jnp.zeros((P, _H), f32)
    for oh in sels:
        G = (jax.lax.dot_general(oh, g_hi, _TN, preferred_element_type=f32)
             + jax.lax.dot_general(oh, g_mid, _TN, preferred_element_type=f32)
             + jax.lax.dot_general(oh, g_lo, _TN, preferred_element_type=f32))
        dk = (G - h).astype(bf)
        m = jnp.maximum(
            (t1 + jnp.dot(dk, W1b, preferred_element_type=f32)) + b1, 0.0)
        S = S + jnp.dot(m.astype(bf), W2, preferred_element_type=f32)
    agg = S + f32(_K) * b2
    return jnp.max(agg.reshape(P // 2, 2, _H), axis=1)


def _edge_multi(hs, P, W1a, W1b, b1, W2, b2):
    f32, bf = np.float32, jnp.bfloat16
    fiota = jax.lax.broadcasted_iota(jnp.int32, (P, P), 0).astype(f32)
    st = []
    for h in hs:
        D, hb = _dmat(h, P)
        st.append({"h": h, "hb": hb, "D": D, "sels": []})
    for _ in range(_K):
        for s in st:
            colmin = jnp.min(s["D"], axis=0, keepdims=True)
            cand = jnp.where(s["D"] == colmin, fiota, f32(P))
            minidx = jnp.min(cand, axis=0, keepdims=True)
            sel = fiota == minidx
            s["sels"].append(sel.astype(bf))
            s["D"] = jnp.where(sel, _BIG, s["D"])
    return [_edge_body(s["h"], s["hb"], P, W1a, W1b, b1, W2, b2, s["sels"])
            for s in st]


def _kernel_a(x_ref, W_in_ref, b_in_ref, W1a0_ref, W1b0_ref, b10_ref,
              W20_ref, b20_ref, h1_ref, d1_ref):
    f32 = np.float32
    hs = []
    for i in range(_G):
        x = x_ref[i].astype(jnp.bfloat16)
        hs.append(jnp.maximum(
            jnp.dot(x, W_in_ref[...], preferred_element_type=f32)
            + b_in_ref[...], 0.0))
    hs = _edge_multi(hs, _P, W1a0_ref[...], W1b0_ref[...], b10_ref[...],
                     W20_ref[...], b20_ref[...])
    for i in range(_G):
        h1_ref[i] = hs[i]
        d1_ref[i], _ = _dmat(hs[i], _P1)


def _sc_topk(d_hbm, o_hbm, kbuf, obuf, sem):
    i32 = jnp.int32
    wid = lax.axis_index("c") * _NTILE + lax.axis_index("s")
    lane = lax.iota(i32, 16)

    def merge(a, av, b, bv):
        br = jnp.flip(b)
        brv = jnp.flip(bv)
        mk = jnp.minimum(a, br)
        mv = jnp.where(a <= br, av, brv)
        return plsc.sort_key_val(mk, mv)

    for gi in range(_GPT):
        g = wid * _GPT + gi
        pltpu.async_copy(d_hbm.at[g], kbuf, sem).wait()

        @pl.loop(0, _P1)
        def _(r):
            pairs = []
            for j in range(_P1 // 16):
                kj = kbuf[r, pl.ds(j * 16, 16)]
                pairs.append(plsc.sort_key_val(kj, lane + j * 16))
            while len(pairs) > 1:
                pairs = [merge(*pairs[i], *pairs[i + 1])
                         for i in range(0, len(pairs), 2)]
            _, vals = pairs[0]
            plsc.store_scatter(obuf, [lane, jnp.full((16,), r, i32)], vals)

        pltpu.async_copy(obuf, o_hbm.at[g], sem).wait()


def _kernel_b(h1_ref, idx_ref, W1a1_ref, W1b1_ref, b11_ref, W21_ref,
              b21_ref, g_ref):
    f32, bf = np.float32, jnp.bfloat16
    fiota = jax.lax.broadcasted_iota(jnp.int32, (_P1, _P1), 0)
    for i in range(_G):
        h = h1_ref[i]
        idx = idx_ref[i]
        sels = [(fiota == idx[k:k + 1, :]).astype(bf) for k in range(_K)]
        hp = _edge_body(h, h.astype(bf), _P1, W1a1_ref[...], W1b1_ref[...],
                        b11_ref[...], W21_ref[...], b21_ref[...], sels)
        g_ref[i] = jnp.max(hp, axis=0, keepdims=True)


def _out_kernel(g_ref, gx_ref, Wg_ref, Wx_ref, bo1_ref, Wo2_ref, bo2_ref,
                Wo3_ref, bo3_ref, o_ref):
    f32, bf = np.float32, jnp.bfloat16
    t = (jnp.dot(g_ref[...].astype(bf), Wg_ref[...], preferred_element_type=f32)
         + jnp.dot(gx_ref[...].astype(bf), Wx_ref[...], preferred_element_type=f32)
         + bo1_ref[...])
    t = jnp.maximum(t, 0.0)
    t = jnp.maximum(
        jnp.dot(t.astype(bf), Wo2_ref[...], preferred_element_type=f32)
        + bo2_ref[...], 0.0)
    o_ref[...] = (jnp.dot(t.astype(bf), Wo3_ref[...], preferred_element_type=f32)
                  + bo3_ref[...])


def _sc_compiler_params():
    cp = pltpu.CompilerParams()
    if "needs_layout_passes" in pltpu.CompilerParams.__dataclass_fields__:
        cp = dataclasses.replace(cp, needs_layout_passes=False)
    return cp


def kernel(x, gx, W_in, b_in, W1_0, b1_0, W2_0, b2_0, W1_1, b1_1, W2_1,
           b2_1, Wo1, bo1, Wo2, bo2, Wo3, bo3):
    f32, bf = np.float32, jnp.bfloat16
    row = lambda v: v.reshape(1, -1).astype(f32)
    full = lambda a: pl.BlockSpec(a.shape, (lambda nd: lambda b: (0,) * nd)(a.ndim))
    hb_ = _B // _NCHUNK

    aconsts = (W_in.astype(bf), row(b_in),
               W1_0[:_H].astype(bf), W1_0[_H:].astype(bf), row(b1_0),
               W2_0.astype(bf), row(b2_0))
    bconsts = (W1_1[:_H].astype(bf), W1_1[_H:].astype(bf), row(b1_1),
               W2_1.astype(bf), row(b2_1))
    mesh = plsc.VectorSubcoreMesh(core_axis_name="c", subcore_axis_name="s")

    h1s, d1s, idxs, gs = [], [], [], []
    for c in range(_NCHUNK):
        xc = jax.lax.slice_in_dim(x, c * hb_, (c + 1) * hb_, axis=0)
        h1, d1 = pl.pallas_call(
            _kernel_a,
            grid=(hb_ // _G,),
            in_specs=[pl.BlockSpec((_G, _P, _DIN), lambda b: (b, 0, 0))]
            + [full(a) for a in aconsts],
            out_specs=[pl.BlockSpec((_G, _P1, _H), lambda b: (b, 0, 0)),
                       pl.BlockSpec((_G, _P1, _P1), lambda b: (b, 0, 0))],
            out_shape=[jax.ShapeDtypeStruct((hb_, _P1, _H), f32),
                       jax.ShapeDtypeStruct((hb_, _P1, _P1), f32)],
        )(xc, *aconsts)
        h1s.append(h1)
        d1s.append(d1)
    for c in range(_NCHUNK):
        idxs.append(pl.kernel(
            _sc_topk,
            out_type=jax.ShapeDtypeStruct((hb_, _K, _P1), jnp.int32),
            mesh=mesh,
            scratch_types=[pltpu.VMEM((_P1, _P1), f32),
                           pltpu.VMEM((_K, _P1), jnp.int32),
                           pltpu.SemaphoreType.DMA],
            compiler_params=_sc_compiler_params(),
        )(d1s[c]))
    for c in range(_NCHUNK):
        g = pl.pallas_call(
            _kernel_b,
            grid=(hb_ // _G,),
            in_specs=[pl.BlockSpec((_G, _P1, _H), lambda b: (b, 0, 0)),
                      pl.BlockSpec((_G, _K, _P1), lambda b: (b, 0, 0))]
            + [full(a) for a in bconsts],
            out_specs=pl.BlockSpec((_G, 1, _H), lambda b: (b, 0, 0)),
            out_shape=jax.ShapeDtypeStruct((hb_, 1, _H), f32),
        )(h1s[c], idxs[c], *bconsts)
        gs.append(g)
    g = jnp.concatenate(gs, axis=0).reshape(_B, _H)

    oconsts = (Wo1[:_H].astype(bf), Wo1[_H:].astype(bf), row(bo1),
               Wo2.astype(bf), row(bo2), Wo3.astype(bf), row(bo3))
    out = pl.pallas_call(
        _out_kernel,
        in_specs=[pl.BlockSpec((_B, _H), lambda: (0, 0)),
                  pl.BlockSpec((_B, _DIN), lambda: (0, 0))]
        + [pl.BlockSpec(a.shape, (lambda nd: lambda: (0,) * nd)(a.ndim))
           for a in oconsts],
        out_specs=pl.BlockSpec((_B, 1), lambda: (0, 0)),
        out_shape=jax.ShapeDtypeStruct((_B, 1), f32),
    )(g, gx, *oconsts)
    return out
